# baseline (speedup 1.0000x reference)
import functools

import jax
import jax.numpy as jnp
from jax import lax
from jax.experimental import pallas as pl
from jax.experimental.pallas import tpu as pltpu
from jax.experimental.pallas import tpu_sc as plsc

BATCH = 16384
DIM = 64

_info = plsc.get_sparse_core_info()
NC, NS, L = _info.num_cores, _info.num_subcores, _info.num_lanes
NW = NC * NS
BPW = BATCH // NW

BN = 32768


def _matvec_body(w_ref, x_ref, o_ref):
    x = x_ref[...]
    wt = w_ref[...].T
    res = jax.lax.dot_general(wt, x, (((1,), (0,)), ((), ())),
                              preferred_element_type=jnp.float32)
    o_ref[...] = res[0]


def _scores(tabT, w_col):
    n = tabT.shape[1]
    grid = (n + BN - 1) // BN
    return pl.pallas_call(
        _matvec_body,
        grid=(grid,),
        in_specs=[
            pl.BlockSpec((DIM, 1), lambda i: (0, 0)),
            pl.BlockSpec((DIM, BN), lambda i: (0, i)),
        ],
        out_specs=pl.BlockSpec((BN,), lambda i: (i,)),
        out_shape=jax.ShapeDtypeStruct((n,), jnp.float32),
    )(w_col, tabT)


NP = 100000
NWIN = NP // 128
NTAIL = NP - NWIN * 128


def _pscore_body(ptabT_hbm, w_hbm, sc_hbm, w_v, win_v, row_v, sem):
    wid = lax.axis_index("s") * NC + lax.axis_index("c")
    pltpu.sync_copy(w_hbm, w_v)
    wvec = [w_v[pl.ds(m * L, L)] for m in range(DIM // L)]

    def do_window(w0, width):
        pltpu.async_copy(ptabT_hbm.at[:, pl.ds(w0, width)],
                         win_v.at[:, pl.ds(0, width)], sem).wait()
        for b in range(width // L):
            acc = jnp.zeros((L,), jnp.float32)
            for c in range(DIM):
                m, i = divmod(c, L)
                wk = wvec[m].at[jnp.full((L,), i, jnp.int32)].get(
                    mode="promise_in_bounds")
                acc = acc + win_v[c, pl.ds(b * L, L)] * wk
            row_v[pl.ds(b * L, L)] = acc
        pltpu.sync_copy(row_v.at[pl.ds(0, width)], sc_hbm.at[pl.ds(w0, width)])

    def step(k, carry):
        w = k * NW + wid

        @pl.when(w < NWIN)
        def _():
            do_window(w * 128, 128)
        return carry

    lax.fori_loop(0, (NWIN + NW - 1) // NW, step, 0)


def _pscores_sc(ptabT, w_flat):
    mesh = plsc.VectorSubcoreMesh(core_axis_name="c", subcore_axis_name="s")
    call = functools.partial(
        pl.kernel,
        mesh=mesh,
        compiler_params=pltpu.CompilerParams(needs_layout_passes=False),
        out_type=jax.ShapeDtypeStruct((NWIN * 128,), jnp.float32),
        scratch_types=[
            pltpu.VMEM((DIM,), jnp.float32),
            pltpu.VMEM((DIM, 128), jnp.float32),
            pltpu.VMEM((128,), jnp.float32),
            pltpu.SemaphoreType.DMA,
        ],
    )(_pscore_body)
    return call(ptabT, w_flat)


def _sc_body(pid_hbm, sid_hbm, sp_hbm, ss_hbm, b_hbm, out_hbm,
             idp_v, ids_v, gp_v, gs_v, b_v, out_v, sem_p, sem_s):
    wid = lax.axis_index("s") * NC + lax.axis_index("c")
    base = wid * BPW

    pltpu.sync_copy(pid_hbm.at[pl.ds(base, BPW)], idp_v)
    pltpu.sync_copy(sid_hbm.at[pl.ds(base, BPW)], ids_v)
    pltpu.sync_copy(b_hbm, b_v)
    cp = pltpu.async_copy(sp_hbm.at[idp_v], gp_v, sem_p)
    cs = pltpu.async_copy(ss_hbm.at[ids_v], gs_v, sem_s)
    cp.wait()
    cs.wait()

    bias = b_v[...]

    def group(g, carry):
        s0 = g * L
        logits = gp_v[pl.ds(s0, L)] + gs_v[pl.ds(s0, L)] + bias
        out_v[pl.ds(s0, L)] = 1.0 / (1.0 + jnp.exp(-logits))
        return carry

    lax.fori_loop(0, BPW // L, group, 0)
    pltpu.sync_copy(out_v, out_hbm.at[pl.ds(base, BPW)])


def _gather_sigmoid(playlist_ids, song_ids, scores_p, scores_s, b_vec):
    mesh = plsc.VectorSubcoreMesh(core_axis_name="c", subcore_axis_name="s")
    call = functools.partial(
        pl.kernel,
        mesh=mesh,
        compiler_params=pltpu.CompilerParams(needs_layout_passes=False),
        out_type=jax.ShapeDtypeStruct((BATCH,), jnp.float32),
        scratch_types=[
            pltpu.VMEM((BPW,), jnp.int32),
            pltpu.VMEM((BPW,), jnp.int32),
            pltpu.VMEM((BPW,), jnp.float32),
            pltpu.VMEM((BPW,), jnp.float32),
            pltpu.VMEM((L,), jnp.float32),
            pltpu.VMEM((BPW,), jnp.float32),
            pltpu.SemaphoreType.DMA,
            pltpu.SemaphoreType.DMA,
        ],
    )(_sc_body)
    return call(playlist_ids, song_ids, scores_p, scores_s, b_vec)


@jax.jit
def _run(playlist_ids, song_ids, ptabT, stabT, fc_w, fc_b):
    w1 = fc_w[:DIM]
    w2 = fc_w[DIM:]
    sp_main = _pscores_sc(ptabT, w1.reshape(DIM))
    sp_tail = _scores(ptabT[:, NWIN * 128:], w1)
    scores_p = jnp.concatenate([sp_main, sp_tail])
    scores_s = _scores(stabT, w2)
    b_vec = jnp.broadcast_to(fc_b.astype(jnp.float32), (L,))
    return _gather_sigmoid(playlist_ids, song_ids, scores_p, scores_s, b_vec)


def kernel(playlist_ids, song_ids, playlist_table, song_table, fc_w, fc_b):
    out = _run(playlist_ids, song_ids, playlist_table.T, song_table.T,
               fc_w, fc_b)
    return out.reshape(BATCH, 1)


# speedup vs baseline: 1.0269x; 1.0269x over previous
import functools

import jax
import jax.numpy as jnp
from jax import lax
from jax.experimental import pallas as pl
from jax.experimental.pallas import tpu as pltpu
from jax.experimental.pallas import tpu_sc as plsc

BATCH = 16384
DIM = 64

_info = plsc.get_sparse_core_info()
NC, NS, L = _info.num_cores, _info.num_subcores, _info.num_lanes
NW = NC * NS
BPW = BATCH // NW

BN = 32768


def _matvec_body(w_ref, x_ref, o_ref):
    x = x_ref[...]
    wt = w_ref[...].T
    res = jax.lax.dot_general(wt, x, (((1,), (0,)), ((), ())),
                              preferred_element_type=jnp.float32)
    o_ref[...] = res[0]


def _scores(tabT, w_col):
    n = tabT.shape[1]
    grid = (n + BN - 1) // BN
    return pl.pallas_call(
        _matvec_body,
        grid=(grid,),
        in_specs=[
            pl.BlockSpec((DIM, 1), lambda i: (0, 0)),
            pl.BlockSpec((DIM, BN), lambda i: (0, i)),
        ],
        out_specs=pl.BlockSpec((BN,), lambda i: (i,)),
        out_shape=jax.ShapeDtypeStruct((n,), jnp.float32),
    )(w_col, tabT)


def _sc_body(pid_hbm, sid_hbm, sp_hbm, ss_hbm, b_hbm, out_hbm,
             idp_v, ids_v, gp_v, gs_v, b_v, out_v, sem_p, sem_s):
    wid = lax.axis_index("s") * NC + lax.axis_index("c")
    base = wid * BPW

    pltpu.sync_copy(pid_hbm.at[pl.ds(base, BPW)], idp_v)
    pltpu.sync_copy(sid_hbm.at[pl.ds(base, BPW)], ids_v)
    pltpu.sync_copy(b_hbm, b_v)
    cp = pltpu.async_copy(sp_hbm.at[idp_v], gp_v, sem_p)
    cs = pltpu.async_copy(ss_hbm.at[ids_v], gs_v, sem_s)
    cp.wait()
    cs.wait()

    bias = b_v[...]

    def group(g, carry):
        s0 = g * L
        logits = gp_v[pl.ds(s0, L)] + gs_v[pl.ds(s0, L)] + bias
        out_v[pl.ds(s0, L)] = 1.0 / (1.0 + jnp.exp(-logits))
        return carry

    lax.fori_loop(0, BPW // L, group, 0)
    pltpu.sync_copy(out_v, out_hbm.at[pl.ds(base, BPW)])


def _gather_sigmoid(playlist_ids, song_ids, scores_p, scores_s, b_vec):
    mesh = plsc.VectorSubcoreMesh(core_axis_name="c", subcore_axis_name="s")
    call = functools.partial(
        pl.kernel,
        mesh=mesh,
        compiler_params=pltpu.CompilerParams(needs_layout_passes=False),
        out_type=jax.ShapeDtypeStruct((BATCH,), jnp.float32),
        scratch_types=[
            pltpu.VMEM((BPW,), jnp.int32),
            pltpu.VMEM((BPW,), jnp.int32),
            pltpu.VMEM((BPW,), jnp.float32),
            pltpu.VMEM((BPW,), jnp.float32),
            pltpu.VMEM((L,), jnp.float32),
            pltpu.VMEM((BPW,), jnp.float32),
            pltpu.SemaphoreType.DMA,
            pltpu.SemaphoreType.DMA,
        ],
    )(_sc_body)
    return call(playlist_ids, song_ids, scores_p, scores_s, b_vec)


@jax.jit
def _run(playlist_ids, song_ids, ptabT, stabT, fc_w, fc_b):
    w1 = fc_w[:DIM]
    w2 = fc_w[DIM:]
    scores_p = _scores(ptabT, w1)
    scores_s = _scores(stabT, w2)
    b_vec = jnp.broadcast_to(fc_b.astype(jnp.float32), (L,))
    return _gather_sigmoid(playlist_ids, song_ids, scores_p, scores_s, b_vec)


def kernel(playlist_ids, song_ids, playlist_table, song_table, fc_w, fc_b):
    out = _run(playlist_ids, song_ids, playlist_table.T, song_table.T,
               fc_w, fc_b)
    return out.reshape(BATCH, 1)
